# baseline (speedup 1.0000x reference)
import functools

import jax
import jax.numpy as jnp
from jax import lax
from jax.experimental import pallas as pl
from jax.experimental.pallas import tpu as pltpu
from jax.experimental.pallas import tpu_sc as plsc

N = 10000
E = 320000
D1 = 128
DH = 64
D2 = 16
NCLS = 8

NC = 2
NS = 16
NW = NC * NS
CH = 40
NCHG = E // CH
K = 10
BATCH = K * CH
RPT = 624
TAIL = N - NS * RPT

_mesh = plsc.VectorSubcoreMesh(core_axis_name="c", subcore_axis_name="s")
_sc_params = pltpu.CompilerParams(use_tc_tiling_on_sc=False)


def _zero_acc(zero_hbm, acc_sh, s):
  r0 = pl.multiple_of(s * RPT, 8)
  pltpu.sync_copy(zero_hbm.at[pl.ds(r0, RPT)], acc_sh.at[pl.ds(r0, RPT)])

  @pl.when(s == 0)
  def _():
    pltpu.sync_copy(zero_hbm.at[pl.ds(NS * RPT, TAIL)],
                    acc_sh.at[pl.ds(NS * RPT, TAIL)])


def _copy_out(acc_sh, outa, outb, c, s):
  r0 = pl.multiple_of(s * RPT, 8)

  @pl.when(c == 0)
  def _():
    pltpu.sync_copy(acc_sh.at[pl.ds(r0, RPT)], outa.at[pl.ds(r0, RPT)])

    @pl.when(s == 0)
    def _():
      pltpu.sync_copy(acc_sh.at[pl.ds(NS * RPT, TAIL)],
                      outa.at[pl.ds(NS * RPT, TAIL)])

  @pl.when(c == 1)
  def _():
    pltpu.sync_copy(acc_sh.at[pl.ds(r0, RPT)], outb.at[pl.ds(r0, RPT)])

    @pl.when(s == 0)
    def _():
      pltpu.sync_copy(acc_sh.at[pl.ds(NS * RPT, TAIL)],
                      outb.at[pl.ds(NS * RPT, TAIL)])


def _run_edge_pass(z_hbm, src_hbm, dst2_hbm, base_chunk, nbatch,
                   sidx3, didx3, bufa, bufb, acc_sh,
                   sem_g, sem_sa, sem_sb, sem_st):
  def fire_stage(t):
    ck = base_chunk + t * K
    pltpu.async_copy(src_hbm.at[pl.ds(pl.multiple_of(ck * CH, 8), BATCH)],
                     sidx3.at[t % 3], sem_st)
    pltpu.async_copy(dst2_hbm.at[pl.ds(ck, K)], didx3.at[t % 3], sem_st)

  def drain_stage(t):
    ck = base_chunk + t * K
    pltpu.make_async_copy(
        src_hbm.at[pl.ds(pl.multiple_of(ck * CH, 8), BATCH)],
        sidx3.at[t % 3], sem_st).wait()
    pltpu.make_async_copy(dst2_hbm.at[pl.ds(ck, K)], didx3.at[t % 3],
                          sem_st).wait()

  def gather_descs(t, buf):
    return [pltpu.make_async_copy(
        z_hbm.at[sidx3.at[t % 3].at[pl.ds(b * CH, CH)]],
        buf.at[pl.ds(b * CH, CH)], sem_g) for b in range(K)]

  def scatter_descs(t, buf, sem):
    return [pltpu.make_async_copy(
        buf.at[pl.ds(b * CH, CH)],
        acc_sh.at[didx3.at[t % 3].at[b]], sem) for b in range(K)]

  def do_batch(t, buf_p, sem_p, buf_q, sem_q):
    for d in gather_descs(t, buf_p):
      d.wait()

    @pl.when(t > 0)
    def _():
      for d in scatter_descs(t - 1, buf_q, sem_q):
        d.wait()

    @pl.when(t + 1 < nbatch)
    def _():
      for d in gather_descs(t + 1, buf_q):
        d.start()

    @pl.when(t + 2 < nbatch)
    def _():
      fire_stage(t + 2)

    for b in range(K):
      pltpu.async_copy(buf_p.at[pl.ds(b * CH, CH)],
                       acc_sh.at[didx3.at[t % 3].at[b]], sem_p, add=True)

  fire_stage(0)
  drain_stage(0)
  if nbatch > 1:
    fire_stage(1)
  for d in gather_descs(0, bufa):
    d.start()

  def body(t, carry):
    @pl.when(t + 1 < nbatch)
    def _():
      drain_stage(t + 1)

    @pl.when(t % 2 == 0)
    def _():
      do_batch(t, bufa, sem_sa, bufb, sem_sb)

    @pl.when(t % 2 == 1)
    def _():
      do_batch(t, bufb, sem_sb, bufa, sem_sa)

    return carry

  lax.fori_loop(0, nbatch, body, 0)
  last = nbatch - 1
  lb, ls = (bufa, sem_sa) if last % 2 == 0 else (bufb, sem_sb)
  for d in scatter_descs(last, lb, ls):
    d.wait()


def _edge_scratch(d):
  return [
      pltpu.VMEM((3, BATCH), jnp.int32),
      pltpu.VMEM((3, K, CH), jnp.int32),
      pltpu.VMEM((BATCH, d), jnp.float32),
      pltpu.VMEM((BATCH, d), jnp.float32),
      pltpu.VMEM_SHARED((N, d), jnp.float32),
      pltpu.SemaphoreType.DMA,
      pltpu.SemaphoreType.DMA,
      pltpu.SemaphoreType.DMA,
      pltpu.SemaphoreType.DMA,
  ]


@functools.partial(
    pl.kernel,
    mesh=_mesh,
    compiler_params=_sc_params,
    out_type=(
        jax.ShapeDtypeStruct((N, DH), jnp.float32),
        jax.ShapeDtypeStruct((N, DH), jnp.float32),
    ),
    scratch_types=_edge_scratch(DH),
)
def _agg64split(za_hbm, zb_hbm, src_hbm, dst2_hbm, zero_hbm, o0, o1,
                sidx3, didx3, bufa, bufb, acc_sh,
                sem_g, sem_sa, sem_sb, sem_st):
  c = lax.axis_index("c")
  s = lax.axis_index("s")
  nbatch = E // NS // BATCH
  base = s * (NCHG // NS)
  _zero_acc(zero_hbm, acc_sh, s)
  plsc.subcore_barrier()

  @pl.when(c == 0)
  def _():
    _run_edge_pass(za_hbm, src_hbm, dst2_hbm, base, nbatch,
                   sidx3, didx3, bufa, bufb, acc_sh,
                   sem_g, sem_sa, sem_sb, sem_st)

  @pl.when(c == 1)
  def _():
    _run_edge_pass(zb_hbm, src_hbm, dst2_hbm, base, nbatch,
                   sidx3, didx3, bufa, bufb, acc_sh,
                   sem_g, sem_sa, sem_sb, sem_st)

  plsc.subcore_barrier()
  _copy_out(acc_sh, o0, o1, c, s)


@functools.partial(
    pl.kernel,
    mesh=_mesh,
    compiler_params=_sc_params,
    out_type=(
        jax.ShapeDtypeStruct((N, D2), jnp.float32),
        jax.ShapeDtypeStruct((N, D2), jnp.float32),
    ),
    scratch_types=_edge_scratch(D2),
)
def _agg16(z_hbm, src_hbm, dst2_hbm, zero_hbm, outa, outb,
           sidx3, didx3, bufa, bufb, acc_sh,
           sem_g, sem_sa, sem_sb, sem_st):
  c = lax.axis_index("c")
  s = lax.axis_index("s")
  wid = s * NC + c
  nbatch = E // NW // BATCH
  base = wid * (NCHG // NW)
  _zero_acc(zero_hbm, acc_sh, s)
  plsc.subcore_barrier()
  _run_edge_pass(z_hbm, src_hbm, dst2_hbm, base, nbatch,
                 sidx3, didx3, bufa, bufb, acc_sh,
                 sem_g, sem_sa, sem_sb, sem_st)
  plsc.subcore_barrier()
  _copy_out(acc_sh, outa, outb, c, s)


@functools.partial(
    pl.kernel,
    mesh=_mesh,
    compiler_params=_sc_params,
    out_type=(
        jax.ShapeDtypeStruct((N, D2), jnp.float32),
        jax.ShapeDtypeStruct((N, D2), jnp.float32),
    ),
    scratch_types=[
        pltpu.VMEM((E // NW // CH, CH), jnp.int32),
        pltpu.VMEM((CH, D2), jnp.float32),
        pltpu.VMEM_SHARED((N, D2), jnp.float32),
        pltpu.SemaphoreType.DMA,
    ],
)
def _deg_kernel(dst2_hbm, ones_hbm, zero_hbm, outa, outb,
                didx_v, ones_v, acc_sh, sem):
  c = lax.axis_index("c")
  s = lax.axis_index("s")
  wid = s * NC + c
  nchunk = E // NW // CH
  pltpu.sync_copy(dst2_hbm.at[pl.ds(wid * nchunk, nchunk)], didx_v)
  pltpu.sync_copy(ones_hbm, ones_v)
  _zero_acc(zero_hbm, acc_sh, s)
  plsc.subcore_barrier()

  def body(t, carry):
    handles = []
    for b in range(K):
      handles.append(
          pltpu.async_copy(ones_v, acc_sh.at[didx_v.at[t * K + b]], sem,
                           add=True))
    for h in handles:
      h.wait()
    return carry

  lax.fori_loop(0, nchunk // K, body, 0)
  plsc.subcore_barrier()
  _copy_out(acc_sh, outa, outb, c, s)


BLK = 1000


def _z1_body(x_ref, w_ref, da_ref, db_ref, oa_ref, ob_ref):
  deg = da_ref[:, 0:1] + db_ref[:, 0:1] + 1.0
  dis = lax.rsqrt(deg)
  z = jnp.dot(x_ref[...], w_ref[...],
              preferred_element_type=jnp.float32) * dis
  oa_ref[...] = z[:, 0:DH]
  ob_ref[...] = z[:, DH:D1]


def _z2_body(s0_ref, s1_ref, za_ref, zb_ref,
             da_ref, db_ref, b1_ref, w2_ref, o_ref):
  deg = da_ref[:, 0:1] + db_ref[:, 0:1] + 1.0
  dis = lax.rsqrt(deg)
  hl = (s0_ref[...] + za_ref[...]) * dis + b1_ref[:, 0:DH]
  hh = (s1_ref[...] + zb_ref[...]) * dis + b1_ref[:, DH:D1]
  h = jnp.maximum(jnp.concatenate([hl, hh], axis=1), 0.0)
  o_ref[...] = jnp.dot(h, w2_ref[...], preferred_element_type=jnp.float32) * dis


def _out_body(sa_ref, sb_ref, z2_ref, da_ref, db_ref, b2_ref, o_ref):
  deg = da_ref[:, 0:1] + db_ref[:, 0:1] + 1.0
  dis = lax.rsqrt(deg)
  y = (sa_ref[...] + sb_ref[...] + z2_ref[...]) * dis
  o_ref[...] = y[:, 0:NCLS] + b2_ref[...]


def _row_spec(d):
  return pl.BlockSpec((BLK, d), lambda i: (i, 0))


def _full_spec(r, c):
  return pl.BlockSpec((r, c), lambda i: (0, 0))


def kernel(x, edge_index, W1, b1, W2, b2):
  src = edge_index[0].astype(jnp.int32)
  dst = edge_index[1].astype(jnp.int32)
  dst2 = dst.reshape(NCHG, CH)
  ones_rows = jnp.ones((CH, D2), jnp.float32)
  zeros16 = jnp.zeros((N, D2), jnp.float32)
  zeros64 = jnp.zeros((N, DH), jnp.float32)
  W2p = jnp.pad(W2, ((0, 0), (0, D2 - NCLS)))
  b1r = b1.reshape(1, D1)
  b2r = b2.reshape(1, NCLS)

  dega, degb = _deg_kernel(dst2, ones_rows, zeros16)

  z1a, z1b = pl.pallas_call(
      _z1_body,
      grid=(N // BLK,),
      in_specs=[_row_spec(D1), _full_spec(D1, D1), _row_spec(D2),
                _row_spec(D2)],
      out_specs=(_row_spec(DH), _row_spec(DH)),
      out_shape=(jax.ShapeDtypeStruct((N, DH), jnp.float32),
                 jax.ShapeDtypeStruct((N, DH), jnp.float32)),
  )(x, W1, dega, degb)

  s0, s1 = _agg64split(z1a, z1b, src, dst2, zeros64)

  z2 = pl.pallas_call(
      _z2_body,
      grid=(N // BLK,),
      in_specs=[_row_spec(DH)] * 4 + [_row_spec(D2), _row_spec(D2),
                _full_spec(1, D1), _full_spec(D1, D2)],
      out_specs=_row_spec(D2),
      out_shape=jax.ShapeDtypeStruct((N, D2), jnp.float32),
  )(s0, s1, z1a, z1b, dega, degb, b1r, W2p)

  s2a, s2b = _agg16(z2, src, dst2, zeros16)

  out = pl.pallas_call(
      _out_body,
      grid=(N // BLK,),
      in_specs=[_row_spec(D2), _row_spec(D2), _row_spec(D2), _row_spec(D2),
                _row_spec(D2), _full_spec(1, NCLS)],
      out_specs=_row_spec(NCLS),
      out_shape=jax.ShapeDtypeStruct((N, NCLS), jnp.float32),
  )(s2a, s2b, z2, dega, degb, b2r)

  return out


# speedup vs baseline: 1.0112x; 1.0112x over previous
import functools

import jax
import jax.numpy as jnp
from jax import lax
from jax.experimental import pallas as pl
from jax.experimental.pallas import tpu as pltpu
from jax.experimental.pallas import tpu_sc as plsc

N = 10000
E = 320000
D1 = 128
DH = 64
D2 = 16
NCLS = 8

NC = 2
NS = 16
NW = NC * NS
CH = 80
NCHG = E // CH
K = 5
BATCH = K * CH
RPT = 624
TAIL = N - NS * RPT

_mesh = plsc.VectorSubcoreMesh(core_axis_name="c", subcore_axis_name="s")
_sc_params = pltpu.CompilerParams(use_tc_tiling_on_sc=False)


def _zero_acc(zero_hbm, acc_sh, s):
  r0 = pl.multiple_of(s * RPT, 8)
  pltpu.sync_copy(zero_hbm.at[pl.ds(r0, RPT)], acc_sh.at[pl.ds(r0, RPT)])

  @pl.when(s == 0)
  def _():
    pltpu.sync_copy(zero_hbm.at[pl.ds(NS * RPT, TAIL)],
                    acc_sh.at[pl.ds(NS * RPT, TAIL)])


def _copy_out(acc_sh, outa, outb, c, s):
  r0 = pl.multiple_of(s * RPT, 8)

  @pl.when(c == 0)
  def _():
    pltpu.sync_copy(acc_sh.at[pl.ds(r0, RPT)], outa.at[pl.ds(r0, RPT)])

    @pl.when(s == 0)
    def _():
      pltpu.sync_copy(acc_sh.at[pl.ds(NS * RPT, TAIL)],
                      outa.at[pl.ds(NS * RPT, TAIL)])

  @pl.when(c == 1)
  def _():
    pltpu.sync_copy(acc_sh.at[pl.ds(r0, RPT)], outb.at[pl.ds(r0, RPT)])

    @pl.when(s == 0)
    def _():
      pltpu.sync_copy(acc_sh.at[pl.ds(NS * RPT, TAIL)],
                      outb.at[pl.ds(NS * RPT, TAIL)])


def _run_edge_pass(z_hbm, src_hbm, dst2_hbm, base_chunk, nbatch,
                   sidx3, didx3, bufa, bufb, acc_sh,
                   sem_g, sem_sa, sem_sb, sem_st):
  def fire_stage(t):
    ck = base_chunk + t * K
    pltpu.async_copy(src_hbm.at[pl.ds(pl.multiple_of(ck * CH, 8), BATCH)],
                     sidx3.at[t % 3], sem_st)
    pltpu.async_copy(dst2_hbm.at[pl.ds(ck, K)], didx3.at[t % 3], sem_st)

  def drain_stage(t):
    ck = base_chunk + t * K
    pltpu.make_async_copy(
        src_hbm.at[pl.ds(pl.multiple_of(ck * CH, 8), BATCH)],
        sidx3.at[t % 3], sem_st).wait()
    pltpu.make_async_copy(dst2_hbm.at[pl.ds(ck, K)], didx3.at[t % 3],
                          sem_st).wait()

  def gather_descs(t, buf):
    return [pltpu.make_async_copy(
        z_hbm.at[sidx3.at[t % 3].at[pl.ds(b * CH, CH)]],
        buf.at[pl.ds(b * CH, CH)], sem_g) for b in range(K)]

  def scatter_descs(t, buf, sem):
    return [pltpu.make_async_copy(
        buf.at[pl.ds(b * CH, CH)],
        acc_sh.at[didx3.at[t % 3].at[b]], sem) for b in range(K)]

  def do_batch(t, buf_p, sem_p, buf_q, sem_q):
    for d in gather_descs(t, buf_p):
      d.wait()

    @pl.when(t > 0)
    def _():
      for d in scatter_descs(t - 1, buf_q, sem_q):
        d.wait()

    @pl.when(t + 1 < nbatch)
    def _():
      for d in gather_descs(t + 1, buf_q):
        d.start()

    @pl.when(t + 2 < nbatch)
    def _():
      fire_stage(t + 2)

    for b in range(K):
      pltpu.async_copy(buf_p.at[pl.ds(b * CH, CH)],
                       acc_sh.at[didx3.at[t % 3].at[b]], sem_p, add=True)

  fire_stage(0)
  drain_stage(0)
  if nbatch > 1:
    fire_stage(1)
  for d in gather_descs(0, bufa):
    d.start()

  def body(t, carry):
    @pl.when(t + 1 < nbatch)
    def _():
      drain_stage(t + 1)

    @pl.when(t % 2 == 0)
    def _():
      do_batch(t, bufa, sem_sa, bufb, sem_sb)

    @pl.when(t % 2 == 1)
    def _():
      do_batch(t, bufb, sem_sb, bufa, sem_sa)

    return carry

  lax.fori_loop(0, nbatch, body, 0)
  last = nbatch - 1
  lb, ls = (bufa, sem_sa) if last % 2 == 0 else (bufb, sem_sb)
  for d in scatter_descs(last, lb, ls):
    d.wait()


def _edge_scratch(d):
  return [
      pltpu.VMEM((3, BATCH), jnp.int32),
      pltpu.VMEM((3, K, CH), jnp.int32),
      pltpu.VMEM((BATCH, d), jnp.float32),
      pltpu.VMEM((BATCH, d), jnp.float32),
      pltpu.VMEM_SHARED((N, d), jnp.float32),
      pltpu.SemaphoreType.DMA,
      pltpu.SemaphoreType.DMA,
      pltpu.SemaphoreType.DMA,
      pltpu.SemaphoreType.DMA,
  ]


@functools.partial(
    pl.kernel,
    mesh=_mesh,
    compiler_params=_sc_params,
    out_type=(
        jax.ShapeDtypeStruct((N, DH), jnp.float32),
        jax.ShapeDtypeStruct((N, DH), jnp.float32),
    ),
    scratch_types=_edge_scratch(DH),
)
def _agg64split(za_hbm, zb_hbm, src_hbm, dst2_hbm, zero_hbm, o0, o1,
                sidx3, didx3, bufa, bufb, acc_sh,
                sem_g, sem_sa, sem_sb, sem_st):
  c = lax.axis_index("c")
  s = lax.axis_index("s")
  nbatch = E // NS // BATCH
  base = s * (NCHG // NS)
  _zero_acc(zero_hbm, acc_sh, s)
  plsc.subcore_barrier()

  @pl.when(c == 0)
  def _():
    _run_edge_pass(za_hbm, src_hbm, dst2_hbm, base, nbatch,
                   sidx3, didx3, bufa, bufb, acc_sh,
                   sem_g, sem_sa, sem_sb, sem_st)

  @pl.when(c == 1)
  def _():
    _run_edge_pass(zb_hbm, src_hbm, dst2_hbm, base, nbatch,
                   sidx3, didx3, bufa, bufb, acc_sh,
                   sem_g, sem_sa, sem_sb, sem_st)

  plsc.subcore_barrier()
  _copy_out(acc_sh, o0, o1, c, s)


@functools.partial(
    pl.kernel,
    mesh=_mesh,
    compiler_params=_sc_params,
    out_type=(
        jax.ShapeDtypeStruct((N, D2), jnp.float32),
        jax.ShapeDtypeStruct((N, D2), jnp.float32),
    ),
    scratch_types=_edge_scratch(D2),
)
def _agg16(z_hbm, src_hbm, dst2_hbm, zero_hbm, outa, outb,
           sidx3, didx3, bufa, bufb, acc_sh,
           sem_g, sem_sa, sem_sb, sem_st):
  c = lax.axis_index("c")
  s = lax.axis_index("s")
  wid = s * NC + c
  nbatch = E // NW // BATCH
  base = wid * (NCHG // NW)
  _zero_acc(zero_hbm, acc_sh, s)
  plsc.subcore_barrier()
  _run_edge_pass(z_hbm, src_hbm, dst2_hbm, base, nbatch,
                 sidx3, didx3, bufa, bufb, acc_sh,
                 sem_g, sem_sa, sem_sb, sem_st)
  plsc.subcore_barrier()
  _copy_out(acc_sh, outa, outb, c, s)


@functools.partial(
    pl.kernel,
    mesh=_mesh,
    compiler_params=_sc_params,
    out_type=(
        jax.ShapeDtypeStruct((N, D2), jnp.float32),
        jax.ShapeDtypeStruct((N, D2), jnp.float32),
    ),
    scratch_types=[
        pltpu.VMEM((E // NW // CH, CH), jnp.int32),
        pltpu.VMEM((CH, D2), jnp.float32),
        pltpu.VMEM_SHARED((N, D2), jnp.float32),
        pltpu.SemaphoreType.DMA,
    ],
)
def _deg_kernel(dst2_hbm, ones_hbm, zero_hbm, outa, outb,
                didx_v, ones_v, acc_sh, sem):
  c = lax.axis_index("c")
  s = lax.axis_index("s")
  wid = s * NC + c
  nchunk = E // NW // CH
  pltpu.sync_copy(dst2_hbm.at[pl.ds(wid * nchunk, nchunk)], didx_v)
  pltpu.sync_copy(ones_hbm, ones_v)
  _zero_acc(zero_hbm, acc_sh, s)
  plsc.subcore_barrier()

  def body(t, carry):
    handles = []
    for b in range(K):
      handles.append(
          pltpu.async_copy(ones_v, acc_sh.at[didx_v.at[t * K + b]], sem,
                           add=True))
    for h in handles:
      h.wait()
    return carry

  lax.fori_loop(0, nchunk // K, body, 0)
  plsc.subcore_barrier()
  _copy_out(acc_sh, outa, outb, c, s)


BLK = 1000


def _p1_body(x_ref, w_ref, o_ref):
  o_ref[...] = jnp.dot(x_ref[...], w_ref[...],
                       preferred_element_type=jnp.float32)


def _z1_body(p_ref, da_ref, db_ref, oa_ref, ob_ref):
  deg = da_ref[:, 0:1] + db_ref[:, 0:1] + 1.0
  dis = lax.rsqrt(deg)
  z = p_ref[...] * dis
  oa_ref[...] = z[:, 0:DH]
  ob_ref[...] = z[:, DH:D1]


def _z2_body(s0_ref, s1_ref, za_ref, zb_ref,
             da_ref, db_ref, b1_ref, w2_ref, o_ref):
  deg = da_ref[:, 0:1] + db_ref[:, 0:1] + 1.0
  dis = lax.rsqrt(deg)
  hl = (s0_ref[...] + za_ref[...]) * dis + b1_ref[:, 0:DH]
  hh = (s1_ref[...] + zb_ref[...]) * dis + b1_ref[:, DH:D1]
  h = jnp.maximum(jnp.concatenate([hl, hh], axis=1), 0.0)
  o_ref[...] = jnp.dot(h, w2_ref[...], preferred_element_type=jnp.float32) * dis


def _out_body(sa_ref, sb_ref, z2_ref, da_ref, db_ref, b2_ref, o_ref):
  deg = da_ref[:, 0:1] + db_ref[:, 0:1] + 1.0
  dis = lax.rsqrt(deg)
  y = (sa_ref[...] + sb_ref[...] + z2_ref[...]) * dis
  o_ref[...] = y[:, 0:NCLS] + b2_ref[...]


def _row_spec(d):
  return pl.BlockSpec((BLK, d), lambda i: (i, 0))


def _full_spec(r, c):
  return pl.BlockSpec((r, c), lambda i: (0, 0))


def kernel(x, edge_index, W1, b1, W2, b2):
  src = edge_index[0].astype(jnp.int32)
  dst = edge_index[1].astype(jnp.int32)
  dst2 = dst.reshape(NCHG, CH)
  ones_rows = jnp.ones((CH, D2), jnp.float32)
  zeros16 = jnp.zeros((N, D2), jnp.float32)
  zeros64 = jnp.zeros((N, DH), jnp.float32)
  W2p = jnp.pad(W2, ((0, 0), (0, D2 - NCLS)))
  b1r = b1.reshape(1, D1)
  b2r = b2.reshape(1, NCLS)

  dega, degb = _deg_kernel(dst2, ones_rows, zeros16)

  p1 = pl.pallas_call(
      _p1_body,
      grid=(N // BLK,),
      in_specs=[_row_spec(D1), _full_spec(D1, D1)],
      out_specs=_row_spec(D1),
      out_shape=jax.ShapeDtypeStruct((N, D1), jnp.float32),
  )(x, W1)

  z1a, z1b = pl.pallas_call(
      _z1_body,
      grid=(N // BLK,),
      in_specs=[_row_spec(D1), _row_spec(D2), _row_spec(D2)],
      out_specs=(_row_spec(DH), _row_spec(DH)),
      out_shape=(jax.ShapeDtypeStruct((N, DH), jnp.float32),
                 jax.ShapeDtypeStruct((N, DH), jnp.float32)),
  )(p1, dega, degb)

  s0, s1 = _agg64split(z1a, z1b, src, dst2, zeros64)

  z2 = pl.pallas_call(
      _z2_body,
      grid=(N // BLK,),
      in_specs=[_row_spec(DH)] * 4 + [_row_spec(D2), _row_spec(D2),
                _full_spec(1, D1), _full_spec(D1, D2)],
      out_specs=_row_spec(D2),
      out_shape=jax.ShapeDtypeStruct((N, D2), jnp.float32),
  )(s0, s1, z1a, z1b, dega, degb, b1r, W2p)

  s2a, s2b = _agg16(z2, src, dst2, zeros16)

  out = pl.pallas_call(
      _out_body,
      grid=(N // BLK,),
      in_specs=[_row_spec(D2), _row_spec(D2), _row_spec(D2), _row_spec(D2),
                _row_spec(D2), _full_spec(1, NCLS)],
      out_specs=_row_spec(NCLS),
      out_shape=jax.ShapeDtypeStruct((N, NCLS), jnp.float32),
  )(s2a, s2b, z2, dega, degb, b2r)

  return out


# speedup vs baseline: 1.0883x; 1.0763x over previous
import functools

import jax
import jax.numpy as jnp
from jax import lax
from jax.experimental import pallas as pl
from jax.experimental.pallas import tpu as pltpu
from jax.experimental.pallas import tpu_sc as plsc

N = 10000
E = 320000
D1 = 128
DH = 64
D2 = 16
NCLS = 8

NC = 2
NS = 16
NW = NC * NS
CH = 80
NCHG = E // CH
K = 5
BATCH = K * CH
RPT = 624
TAIL = N - NS * RPT

_mesh = plsc.VectorSubcoreMesh(core_axis_name="c", subcore_axis_name="s")
_sc_params = pltpu.CompilerParams(use_tc_tiling_on_sc=False)


def _zero_acc(zero_hbm, acc_sh, s):
  r0 = pl.multiple_of(s * RPT, 8)
  pltpu.sync_copy(zero_hbm.at[pl.ds(r0, RPT)], acc_sh.at[pl.ds(r0, RPT)])

  @pl.when(s == 0)
  def _():
    pltpu.sync_copy(zero_hbm.at[pl.ds(NS * RPT, TAIL)],
                    acc_sh.at[pl.ds(NS * RPT, TAIL)])


def _copy_out(acc_sh, outa, outb, c, s):
  r0 = pl.multiple_of(s * RPT, 8)

  @pl.when(c == 0)
  def _():
    pltpu.sync_copy(acc_sh.at[pl.ds(r0, RPT)], outa.at[pl.ds(r0, RPT)])

    @pl.when(s == 0)
    def _():
      pltpu.sync_copy(acc_sh.at[pl.ds(NS * RPT, TAIL)],
                      outa.at[pl.ds(NS * RPT, TAIL)])

  @pl.when(c == 1)
  def _():
    pltpu.sync_copy(acc_sh.at[pl.ds(r0, RPT)], outb.at[pl.ds(r0, RPT)])

    @pl.when(s == 0)
    def _():
      pltpu.sync_copy(acc_sh.at[pl.ds(NS * RPT, TAIL)],
                      outb.at[pl.ds(NS * RPT, TAIL)])


def _run_edge_pass(z_hbm, src_hbm, dst2_hbm, base_chunk, nbatch,
                   sidx3, didx3, bufa, bufb, acc_sh,
                   sem_ga, sem_gb, sem_sa, sem_sb, sem_st):
  def fire_stage(t):
    ck = base_chunk + t * K
    pltpu.async_copy(src_hbm.at[pl.ds(pl.multiple_of(ck * CH, 8), BATCH)],
                     sidx3.at[t % 3], sem_st)
    pltpu.async_copy(dst2_hbm.at[pl.ds(ck, K)], didx3.at[t % 3], sem_st)

  def drain_stage(t):
    ck = base_chunk + t * K
    pltpu.make_async_copy(
        src_hbm.at[pl.ds(pl.multiple_of(ck * CH, 8), BATCH)],
        sidx3.at[t % 3], sem_st).wait()
    pltpu.make_async_copy(dst2_hbm.at[pl.ds(ck, K)], didx3.at[t % 3],
                          sem_st).wait()

  def gather_descs(t, buf, sem):
    return [pltpu.make_async_copy(
        z_hbm.at[sidx3.at[t % 3].at[pl.ds(b * CH, CH)]],
        buf.at[pl.ds(b * CH, CH)], sem) for b in range(K)]

  def scatter_descs(t, buf, sem):
    return [pltpu.make_async_copy(
        buf.at[pl.ds(b * CH, CH)],
        acc_sh.at[didx3.at[t % 3].at[b]], sem) for b in range(K)]

  def do_batch(t, buf_p, sem_gp, sem_sp, buf_q, sem_gq, sem_sq):
    @pl.when(t > 0)
    def _():
      for d in scatter_descs(t - 1, buf_q, sem_sq):
        d.wait()

    @pl.when(t + 1 < nbatch)
    def _():
      for d in gather_descs(t + 1, buf_q, sem_gq):
        d.start()

    for d in gather_descs(t, buf_p, sem_gp):
      d.wait()
    for b in range(K):
      pltpu.async_copy(buf_p.at[pl.ds(b * CH, CH)],
                       acc_sh.at[didx3.at[t % 3].at[b]], sem_sp, add=True)

    @pl.when(t + 2 < nbatch)
    def _():
      fire_stage(t + 2)

  fire_stage(0)
  drain_stage(0)
  if nbatch > 1:
    fire_stage(1)
  for d in gather_descs(0, bufa, sem_ga):
    d.start()

  def body(t, carry):
    @pl.when(t + 1 < nbatch)
    def _():
      drain_stage(t + 1)

    @pl.when(t % 2 == 0)
    def _():
      do_batch(t, bufa, sem_ga, sem_sa, bufb, sem_gb, sem_sb)

    @pl.when(t % 2 == 1)
    def _():
      do_batch(t, bufb, sem_gb, sem_sb, bufa, sem_ga, sem_sa)

    return carry

  lax.fori_loop(0, nbatch, body, 0)
  last = nbatch - 1
  lb, ls = (bufa, sem_sa) if last % 2 == 0 else (bufb, sem_sb)
  for d in scatter_descs(last, lb, ls):
    d.wait()


def _edge_scratch(d):
  return [
      pltpu.VMEM((3, BATCH), jnp.int32),
      pltpu.VMEM((3, K, CH), jnp.int32),
      pltpu.VMEM((BATCH, d), jnp.float32),
      pltpu.VMEM((BATCH, d), jnp.float32),
      pltpu.VMEM_SHARED((N, d), jnp.float32),
      pltpu.SemaphoreType.DMA,
      pltpu.SemaphoreType.DMA,
      pltpu.SemaphoreType.DMA,
      pltpu.SemaphoreType.DMA,
      pltpu.SemaphoreType.DMA,
  ]


@functools.partial(
    pl.kernel,
    mesh=_mesh,
    compiler_params=_sc_params,
    out_type=(
        jax.ShapeDtypeStruct((N, DH), jnp.float32),
        jax.ShapeDtypeStruct((N, DH), jnp.float32),
    ),
    scratch_types=_edge_scratch(DH),
)
def _agg64split(za_hbm, zb_hbm, src_hbm, dst2_hbm, zero_hbm, o0, o1,
                sidx3, didx3, bufa, bufb, acc_sh,
                sem_ga, sem_gb, sem_sa, sem_sb, sem_st):
  c = lax.axis_index("c")
  s = lax.axis_index("s")
  nbatch = E // NS // BATCH
  base = s * (NCHG // NS)
  _zero_acc(zero_hbm, acc_sh, s)
  plsc.subcore_barrier()

  @pl.when(c == 0)
  def _():
    _run_edge_pass(za_hbm, src_hbm, dst2_hbm, base, nbatch,
                   sidx3, didx3, bufa, bufb, acc_sh,
                   sem_ga, sem_gb, sem_sa, sem_sb, sem_st)

  @pl.when(c == 1)
  def _():
    _run_edge_pass(zb_hbm, src_hbm, dst2_hbm, base, nbatch,
                   sidx3, didx3, bufa, bufb, acc_sh,
                   sem_ga, sem_gb, sem_sa, sem_sb, sem_st)

  plsc.subcore_barrier()
  _copy_out(acc_sh, o0, o1, c, s)


@functools.partial(
    pl.kernel,
    mesh=_mesh,
    compiler_params=_sc_params,
    out_type=(
        jax.ShapeDtypeStruct((N, D2), jnp.float32),
        jax.ShapeDtypeStruct((N, D2), jnp.float32),
    ),
    scratch_types=_edge_scratch(D2),
)
def _agg16(z_hbm, src_hbm, dst2_hbm, zero_hbm, outa, outb,
           sidx3, didx3, bufa, bufb, acc_sh,
           sem_ga, sem_gb, sem_sa, sem_sb, sem_st):
  c = lax.axis_index("c")
  s = lax.axis_index("s")
  wid = s * NC + c
  nbatch = E // NW // BATCH
  base = wid * (NCHG // NW)
  _zero_acc(zero_hbm, acc_sh, s)
  plsc.subcore_barrier()
  _run_edge_pass(z_hbm, src_hbm, dst2_hbm, base, nbatch,
                 sidx3, didx3, bufa, bufb, acc_sh,
                 sem_ga, sem_gb, sem_sa, sem_sb, sem_st)
  plsc.subcore_barrier()
  _copy_out(acc_sh, outa, outb, c, s)


@functools.partial(
    pl.kernel,
    mesh=_mesh,
    compiler_params=_sc_params,
    out_type=(
        jax.ShapeDtypeStruct((N, D2), jnp.float32),
        jax.ShapeDtypeStruct((N, D2), jnp.float32),
    ),
    scratch_types=[
        pltpu.VMEM((E // NW // CH, CH), jnp.int32),
        pltpu.VMEM((CH, D2), jnp.float32),
        pltpu.VMEM_SHARED((N, D2), jnp.float32),
        pltpu.SemaphoreType.DMA,
    ],
)
def _deg_kernel(dst2_hbm, ones_hbm, zero_hbm, outa, outb,
                didx_v, ones_v, acc_sh, sem):
  c = lax.axis_index("c")
  s = lax.axis_index("s")
  wid = s * NC + c
  nchunk = E // NW // CH
  pltpu.sync_copy(dst2_hbm.at[pl.ds(wid * nchunk, nchunk)], didx_v)
  pltpu.sync_copy(ones_hbm, ones_v)
  _zero_acc(zero_hbm, acc_sh, s)
  plsc.subcore_barrier()

  def body(t, carry):
    handles = []
    for b in range(K):
      handles.append(
          pltpu.async_copy(ones_v, acc_sh.at[didx_v.at[t * K + b]], sem,
                           add=True))
    for h in handles:
      h.wait()
    return carry

  lax.fori_loop(0, nchunk // K, body, 0)
  plsc.subcore_barrier()
  _copy_out(acc_sh, outa, outb, c, s)


BLK = 1000


def _z1_body(x_ref, w_ref, da_ref, db_ref, oa_ref, ob_ref):
  deg = da_ref[:, 0:1] + db_ref[:, 0:1] + 1.0
  dis = lax.rsqrt(deg)
  z = jnp.dot(x_ref[...], w_ref[...],
              preferred_element_type=jnp.float32) * dis
  oa_ref[...] = z[:, 0:DH]
  ob_ref[...] = z[:, DH:D1]


def _z2_body(s0_ref, s1_ref, za_ref, zb_ref,
             da_ref, db_ref, b1_ref, w2_ref, o_ref):
  deg = da_ref[:, 0:1] + db_ref[:, 0:1] + 1.0
  dis = lax.rsqrt(deg)
  hl = (s0_ref[...] + za_ref[...]) * dis + b1_ref[:, 0:DH]
  hh = (s1_ref[...] + zb_ref[...]) * dis + b1_ref[:, DH:D1]
  h = jnp.maximum(jnp.concatenate([hl, hh], axis=1), 0.0)
  o_ref[...] = jnp.dot(h, w2_ref[...], preferred_element_type=jnp.float32) * dis


def _out_body(sa_ref, sb_ref, z2_ref, da_ref, db_ref, b2_ref, o_ref):
  deg = da_ref[:, 0:1] + db_ref[:, 0:1] + 1.0
  dis = lax.rsqrt(deg)
  y = (sa_ref[...] + sb_ref[...] + z2_ref[...]) * dis
  o_ref[...] = y[:, 0:NCLS] + b2_ref[...]


def _row_spec(d):
  return pl.BlockSpec((BLK, d), lambda i: (i, 0))


def _full_spec(r, c):
  return pl.BlockSpec((r, c), lambda i: (0, 0))


def kernel(x, edge_index, W1, b1, W2, b2):
  src = edge_index[0].astype(jnp.int32)
  dst = edge_index[1].astype(jnp.int32)
  dst2 = dst.reshape(NCHG, CH)
  ones_rows = jnp.ones((CH, D2), jnp.float32)
  zeros16 = jnp.zeros((N, D2), jnp.float32)
  zeros64 = jnp.zeros((N, DH), jnp.float32)
  W2p = jnp.pad(W2, ((0, 0), (0, D2 - NCLS)))
  b1r = b1.reshape(1, D1)
  b2r = b2.reshape(1, NCLS)

  dega, degb = _deg_kernel(dst2, ones_rows, zeros16)

  z1a, z1b = pl.pallas_call(
      _z1_body,
      grid=(N // BLK,),
      in_specs=[_row_spec(D1), _full_spec(D1, D1), _row_spec(D2),
                _row_spec(D2)],
      out_specs=(_row_spec(DH), _row_spec(DH)),
      out_shape=(jax.ShapeDtypeStruct((N, DH), jnp.float32),
                 jax.ShapeDtypeStruct((N, DH), jnp.float32)),
  )(x, W1, dega, degb)

  s0, s1 = _agg64split(z1a, z1b, src, dst2, zeros64)

  z2 = pl.pallas_call(
      _z2_body,
      grid=(N // BLK,),
      in_specs=[_row_spec(DH)] * 4 + [_row_spec(D2), _row_spec(D2),
                _full_spec(1, D1), _full_spec(D1, D2)],
      out_specs=_row_spec(D2),
      out_shape=jax.ShapeDtypeStruct((N, D2), jnp.float32),
  )(s0, s1, z1a, z1b, dega, degb, b1r, W2p)

  s2a, s2b = _agg16(z2, src, dst2, zeros16)

  out = pl.pallas_call(
      _out_body,
      grid=(N // BLK,),
      in_specs=[_row_spec(D2), _row_spec(D2), _row_spec(D2), _row_spec(D2),
                _row_spec(D2), _full_spec(1, NCLS)],
      out_specs=_row_spec(NCLS),
      out_shape=jax.ShapeDtypeStruct((N, NCLS), jnp.float32),
  )(s2a, s2b, z2, dega, degb, b2r)

  return out


# speedup vs baseline: 1.0912x; 1.0026x over previous
import functools

import jax
import jax.numpy as jnp
from jax import lax
from jax.experimental import pallas as pl
from jax.experimental.pallas import tpu as pltpu
from jax.experimental.pallas import tpu_sc as plsc

N = 10000
E = 320000
D1 = 128
DH = 64
D2 = 16
NCLS = 8

NC = 2
NS = 16
NW = NC * NS
CH = 80
NCHG = E // CH
K = 5
BATCH = K * CH
RPT = 624
TAIL = N - NS * RPT

_mesh = plsc.VectorSubcoreMesh(core_axis_name="c", subcore_axis_name="s")
_sc_params = pltpu.CompilerParams(use_tc_tiling_on_sc=False)


def _zero_acc(zero_hbm, acc_sh, s):
  r0 = pl.multiple_of(s * RPT, 8)
  pltpu.sync_copy(zero_hbm.at[pl.ds(r0, RPT)], acc_sh.at[pl.ds(r0, RPT)])

  @pl.when(s == 0)
  def _():
    pltpu.sync_copy(zero_hbm.at[pl.ds(NS * RPT, TAIL)],
                    acc_sh.at[pl.ds(NS * RPT, TAIL)])


def _copy_out(acc_sh, outa, outb, c, s):
  r0 = pl.multiple_of(s * RPT, 8)

  @pl.when(c == 0)
  def _():
    pltpu.sync_copy(acc_sh.at[pl.ds(r0, RPT)], outa.at[pl.ds(r0, RPT)])

    @pl.when(s == 0)
    def _():
      pltpu.sync_copy(acc_sh.at[pl.ds(NS * RPT, TAIL)],
                      outa.at[pl.ds(NS * RPT, TAIL)])

  @pl.when(c == 1)
  def _():
    pltpu.sync_copy(acc_sh.at[pl.ds(r0, RPT)], outb.at[pl.ds(r0, RPT)])

    @pl.when(s == 0)
    def _():
      pltpu.sync_copy(acc_sh.at[pl.ds(NS * RPT, TAIL)],
                      outb.at[pl.ds(NS * RPT, TAIL)])


def _run_edge_pass(z_hbm, src_hbm, dst2_hbm, base_chunk, nbatch,
                   sidx3, didx3, bufa, bufb, acc_sh,
                   sem_ga, sem_gb, sem_sa, sem_sb, sem_st):
  def fire_stage(t):
    ck = base_chunk + t * K
    pltpu.async_copy(src_hbm.at[pl.ds(pl.multiple_of(ck * CH, 8), BATCH)],
                     sidx3.at[t % 3], sem_st)
    pltpu.async_copy(dst2_hbm.at[pl.ds(ck, K)], didx3.at[t % 3], sem_st)

  def drain_stage(t):
    ck = base_chunk + t * K
    pltpu.make_async_copy(
        src_hbm.at[pl.ds(pl.multiple_of(ck * CH, 8), BATCH)],
        sidx3.at[t % 3], sem_st).wait()
    pltpu.make_async_copy(dst2_hbm.at[pl.ds(ck, K)], didx3.at[t % 3],
                          sem_st).wait()

  def gather_descs(t, buf, sem):
    return [pltpu.make_async_copy(
        z_hbm.at[sidx3.at[t % 3].at[pl.ds(b * CH, CH)]],
        buf.at[pl.ds(b * CH, CH)], sem) for b in range(K)]

  def scatter_descs(t, buf, sem):
    return [pltpu.make_async_copy(
        buf.at[pl.ds(b * CH, CH)],
        acc_sh.at[didx3.at[t % 3].at[b]], sem) for b in range(K)]

  def do_batch(t, buf_p, sem_gp, sem_sp, buf_q, sem_gq, sem_sq):
    @pl.when(t > 0)
    def _():
      for d in scatter_descs(t - 1, buf_q, sem_sq):
        d.wait()

    @pl.when(t + 1 < nbatch)
    def _():
      for d in gather_descs(t + 1, buf_q, sem_gq):
        d.start()

    for d in gather_descs(t, buf_p, sem_gp):
      d.wait()
    for b in range(K):
      pltpu.async_copy(buf_p.at[pl.ds(b * CH, CH)],
                       acc_sh.at[didx3.at[t % 3].at[b]], sem_sp, add=True)

    @pl.when(t + 2 < nbatch)
    def _():
      fire_stage(t + 2)

  fire_stage(0)
  drain_stage(0)
  if nbatch > 1:
    fire_stage(1)
  for d in gather_descs(0, bufa, sem_ga):
    d.start()

  def body(t, carry):
    @pl.when(t + 1 < nbatch)
    def _():
      drain_stage(t + 1)

    @pl.when(t % 2 == 0)
    def _():
      do_batch(t, bufa, sem_ga, sem_sa, bufb, sem_gb, sem_sb)

    @pl.when(t % 2 == 1)
    def _():
      do_batch(t, bufb, sem_gb, sem_sb, bufa, sem_ga, sem_sa)

    return carry

  lax.fori_loop(0, nbatch, body, 0)
  last = nbatch - 1
  lb, ls = (bufa, sem_sa) if last % 2 == 0 else (bufb, sem_sb)
  for d in scatter_descs(last, lb, ls):
    d.wait()


def _edge_scratch(d):
  return [
      pltpu.VMEM((3, BATCH), jnp.int32),
      pltpu.VMEM((3, K, CH), jnp.int32),
      pltpu.VMEM((BATCH, d), jnp.float32),
      pltpu.VMEM((BATCH, d), jnp.float32),
      pltpu.VMEM_SHARED((N, d), jnp.float32),
      pltpu.SemaphoreType.DMA,
      pltpu.SemaphoreType.DMA,
      pltpu.SemaphoreType.DMA,
      pltpu.SemaphoreType.DMA,
      pltpu.SemaphoreType.DMA,
  ]


@functools.partial(
    pl.kernel,
    mesh=_mesh,
    compiler_params=_sc_params,
    out_type=(
        jax.ShapeDtypeStruct((N, DH), jnp.float32),
        jax.ShapeDtypeStruct((N, DH), jnp.float32),
    ),
    scratch_types=_edge_scratch(DH),
)
def _agg64split(za_hbm, zb_hbm, src_hbm, dst2_hbm, zero_hbm, o0, o1,
                sidx3, didx3, bufa, bufb, acc_sh,
                sem_ga, sem_gb, sem_sa, sem_sb, sem_st):
  c = lax.axis_index("c")
  s = lax.axis_index("s")
  nbatch = E // NS // BATCH
  base = s * (NCHG // NS)
  _zero_acc(zero_hbm, acc_sh, s)
  plsc.subcore_barrier()

  @pl.when(c == 0)
  def _():
    _run_edge_pass(za_hbm, src_hbm, dst2_hbm, base, nbatch,
                   sidx3, didx3, bufa, bufb, acc_sh,
                   sem_ga, sem_gb, sem_sa, sem_sb, sem_st)

  @pl.when(c == 1)
  def _():
    _run_edge_pass(zb_hbm, src_hbm, dst2_hbm, base, nbatch,
                   sidx3, didx3, bufa, bufb, acc_sh,
                   sem_ga, sem_gb, sem_sa, sem_sb, sem_st)

  plsc.subcore_barrier()
  _copy_out(acc_sh, o0, o1, c, s)


@functools.partial(
    pl.kernel,
    mesh=_mesh,
    compiler_params=_sc_params,
    out_type=(
        jax.ShapeDtypeStruct((N, D2), jnp.float32),
        jax.ShapeDtypeStruct((N, D2), jnp.float32),
    ),
    scratch_types=_edge_scratch(D2),
)
def _agg16(z_hbm, src_hbm, dst2_hbm, zero_hbm, outa, outb,
           sidx3, didx3, bufa, bufb, acc_sh,
           sem_ga, sem_gb, sem_sa, sem_sb, sem_st):
  c = lax.axis_index("c")
  s = lax.axis_index("s")
  wid = s * NC + c
  nbatch = E // NW // BATCH
  base = wid * (NCHG // NW)
  _zero_acc(zero_hbm, acc_sh, s)
  plsc.subcore_barrier()
  _run_edge_pass(z_hbm, src_hbm, dst2_hbm, base, nbatch,
                 sidx3, didx3, bufa, bufb, acc_sh,
                 sem_ga, sem_gb, sem_sa, sem_sb, sem_st)
  plsc.subcore_barrier()
  _copy_out(acc_sh, outa, outb, c, s)


@functools.partial(
    pl.kernel,
    mesh=_mesh,
    compiler_params=_sc_params,
    out_type=(
        jax.ShapeDtypeStruct((N, D2), jnp.float32),
        jax.ShapeDtypeStruct((N, D2), jnp.float32),
    ),
    scratch_types=[
        pltpu.VMEM((E // NW // CH, CH), jnp.int32),
        pltpu.VMEM((CH, D2), jnp.float32),
        pltpu.VMEM_SHARED((N, D2), jnp.float32),
        pltpu.SemaphoreType.DMA,
        pltpu.SemaphoreType.DMA,
    ],
)
def _deg_kernel(dst2_hbm, ones_hbm, zero_hbm, outa, outb,
                didx_v, ones_v, acc_sh, sem_a, sem_b):
  c = lax.axis_index("c")
  s = lax.axis_index("s")
  wid = s * NC + c
  nchunk = E // NW // CH
  nb = nchunk // K
  pltpu.sync_copy(dst2_hbm.at[pl.ds(wid * nchunk, nchunk)], didx_v)
  pltpu.sync_copy(ones_hbm, ones_v)
  _zero_acc(zero_hbm, acc_sh, s)
  plsc.subcore_barrier()

  def descs(t, sem):
    return [pltpu.make_async_copy(ones_v, acc_sh.at[didx_v.at[t * K + b]],
                                  sem) for b in range(K)]

  def half(t, sem_p, sem_q):
    for b in range(K):
      pltpu.async_copy(ones_v, acc_sh.at[didx_v.at[t * K + b]], sem_p,
                       add=True)

    @pl.when(t > 0)
    def _():
      for d in descs(t - 1, sem_q):
        d.wait()

  def body(t, carry):
    @pl.when(t % 2 == 0)
    def _():
      half(t, sem_a, sem_b)

    @pl.when(t % 2 == 1)
    def _():
      half(t, sem_b, sem_a)

    return carry

  lax.fori_loop(0, nb, body, 0)
  for d in descs(nb - 1, sem_a if (nb - 1) % 2 == 0 else sem_b):
    d.wait()
  plsc.subcore_barrier()
  _copy_out(acc_sh, outa, outb, c, s)


BLK = 1000


def _z1_body(x_ref, w_ref, da_ref, db_ref, oa_ref, ob_ref):
  deg = da_ref[:, 0:1] + db_ref[:, 0:1] + 1.0
  dis = lax.rsqrt(deg)
  z = jnp.dot(x_ref[...], w_ref[...],
              preferred_element_type=jnp.float32) * dis
  oa_ref[...] = z[:, 0:DH]
  ob_ref[...] = z[:, DH:D1]


def _z2_body(s0_ref, s1_ref, za_ref, zb_ref,
             da_ref, db_ref, b1_ref, w2_ref, o_ref):
  deg = da_ref[:, 0:1] + db_ref[:, 0:1] + 1.0
  dis = lax.rsqrt(deg)
  hl = (s0_ref[...] + za_ref[...]) * dis + b1_ref[:, 0:DH]
  hh = (s1_ref[...] + zb_ref[...]) * dis + b1_ref[:, DH:D1]
  h = jnp.maximum(jnp.concatenate([hl, hh], axis=1), 0.0)
  o_ref[...] = jnp.dot(h, w2_ref[...], preferred_element_type=jnp.float32) * dis


def _out_body(sa_ref, sb_ref, z2_ref, da_ref, db_ref, b2_ref, o_ref):
  deg = da_ref[:, 0:1] + db_ref[:, 0:1] + 1.0
  dis = lax.rsqrt(deg)
  y = (sa_ref[...] + sb_ref[...] + z2_ref[...]) * dis
  o_ref[...] = y[:, 0:NCLS] + b2_ref[...]


def _row_spec(d):
  return pl.BlockSpec((BLK, d), lambda i: (i, 0))


def _full_spec(r, c):
  return pl.BlockSpec((r, c), lambda i: (0, 0))


def kernel(x, edge_index, W1, b1, W2, b2):
  src = edge_index[0].astype(jnp.int32)
  dst = edge_index[1].astype(jnp.int32)
  dst2 = dst.reshape(NCHG, CH)
  ones_rows = jnp.ones((CH, D2), jnp.float32)
  zeros16 = jnp.zeros((N, D2), jnp.float32)
  zeros64 = jnp.zeros((N, DH), jnp.float32)
  W2p = jnp.pad(W2, ((0, 0), (0, D2 - NCLS)))
  b1r = b1.reshape(1, D1)
  b2r = b2.reshape(1, NCLS)

  dega, degb = _deg_kernel(dst2, ones_rows, zeros16)

  z1a, z1b = pl.pallas_call(
      _z1_body,
      grid=(N // BLK,),
      in_specs=[_row_spec(D1), _full_spec(D1, D1), _row_spec(D2),
                _row_spec(D2)],
      out_specs=(_row_spec(DH), _row_spec(DH)),
      out_shape=(jax.ShapeDtypeStruct((N, DH), jnp.float32),
                 jax.ShapeDtypeStruct((N, DH), jnp.float32)),
  )(x, W1, dega, degb)

  s0, s1 = _agg64split(z1a, z1b, src, dst2, zeros64)

  z2 = pl.pallas_call(
      _z2_body,
      grid=(N // BLK,),
      in_specs=[_row_spec(DH)] * 4 + [_row_spec(D2), _row_spec(D2),
                _full_spec(1, D1), _full_spec(D1, D2)],
      out_specs=_row_spec(D2),
      out_shape=jax.ShapeDtypeStruct((N, D2), jnp.float32),
  )(s0, s1, z1a, z1b, dega, degb, b1r, W2p)

  s2a, s2b = _agg16(z2, src, dst2, zeros16)

  out = pl.pallas_call(
      _out_body,
      grid=(N // BLK,),
      in_specs=[_row_spec(D2), _row_spec(D2), _row_spec(D2), _row_spec(D2),
                _row_spec(D2), _full_spec(1, NCLS)],
      out_specs=_row_spec(NCLS),
      out_shape=jax.ShapeDtypeStruct((N, NCLS), jnp.float32),
  )(s2a, s2b, z2, dega, degb, b2r)

  return out


# speedup vs baseline: 1.1089x; 1.0162x over previous
import functools

import jax
import jax.numpy as jnp
from jax import lax
from jax.experimental import pallas as pl
from jax.experimental.pallas import tpu as pltpu
from jax.experimental.pallas import tpu_sc as plsc

N = 10000
E = 320000
D1 = 128
DH = 64
D2 = 16
NCLS = 8

NC = 2
NS = 16
NW = NC * NS
CH = 80
NCHG = E // CH
K = 5
BATCH = K * CH
RPT = 624
TAIL = N - NS * RPT

_mesh = plsc.VectorSubcoreMesh(core_axis_name="c", subcore_axis_name="s")
_sc_params = pltpu.CompilerParams(use_tc_tiling_on_sc=False)


def _zero_acc(zero_hbm, acc_sh, s):
  r0 = pl.multiple_of(s * RPT, 8)
  pltpu.sync_copy(zero_hbm.at[pl.ds(r0, RPT)], acc_sh.at[pl.ds(r0, RPT)])

  @pl.when(s == 0)
  def _():
    pltpu.sync_copy(zero_hbm.at[pl.ds(NS * RPT, TAIL)],
                    acc_sh.at[pl.ds(NS * RPT, TAIL)])


def _copy_out(acc_sh, outa, outb, c, s):
  r0 = pl.multiple_of(s * RPT, 8)

  @pl.when(c == 0)
  def _():
    pltpu.sync_copy(acc_sh.at[pl.ds(r0, RPT)], outa.at[pl.ds(r0, RPT)])

    @pl.when(s == 0)
    def _():
      pltpu.sync_copy(acc_sh.at[pl.ds(NS * RPT, TAIL)],
                      outa.at[pl.ds(NS * RPT, TAIL)])

  @pl.when(c == 1)
  def _():
    pltpu.sync_copy(acc_sh.at[pl.ds(r0, RPT)], outb.at[pl.ds(r0, RPT)])

    @pl.when(s == 0)
    def _():
      pltpu.sync_copy(acc_sh.at[pl.ds(NS * RPT, TAIL)],
                      outb.at[pl.ds(NS * RPT, TAIL)])


def _run_edge_pass(z_hbm, src_hbm, dst2_hbm, base_chunk, nbatch,
                   sidx3, didx3, bufa, bufb, acc_sh,
                   sem_ga, sem_gb, sem_sa, sem_sb, sem_st):
  def fire_stage(t):
    ck = base_chunk + t * K
    pltpu.async_copy(src_hbm.at[pl.ds(pl.multiple_of(ck * CH, 8), BATCH)],
                     sidx3.at[t % 3], sem_st)
    pltpu.async_copy(dst2_hbm.at[pl.ds(ck, K)], didx3.at[t % 3], sem_st)

  def drain_stage(t):
    ck = base_chunk + t * K
    pltpu.make_async_copy(
        src_hbm.at[pl.ds(pl.multiple_of(ck * CH, 8), BATCH)],
        sidx3.at[t % 3], sem_st).wait()
    pltpu.make_async_copy(dst2_hbm.at[pl.ds(ck, K)], didx3.at[t % 3],
                          sem_st).wait()

  def gather_descs(t, buf, sem):
    return [pltpu.make_async_copy(
        z_hbm.at[sidx3.at[t % 3].at[pl.ds(b * CH, CH)]],
        buf.at[pl.ds(b * CH, CH)], sem) for b in range(K)]

  def scatter_descs(t, buf, sem):
    return [pltpu.make_async_copy(
        buf.at[pl.ds(b * CH, CH)],
        acc_sh.at[didx3.at[t % 3].at[b]], sem) for b in range(K)]

  def do_batch(t, buf_p, sem_gp, sem_sp, buf_q, sem_gq, sem_sq):
    @pl.when(t > 0)
    def _():
      for d in scatter_descs(t - 1, buf_q, sem_sq):
        d.wait()

    @pl.when(t + 1 < nbatch)
    def _():
      for d in gather_descs(t + 1, buf_q, sem_gq):
        d.start()

    for d in gather_descs(t, buf_p, sem_gp):
      d.wait()
    for b in range(K):
      pltpu.async_copy(buf_p.at[pl.ds(b * CH, CH)],
                       acc_sh.at[didx3.at[t % 3].at[b]], sem_sp, add=True)

    @pl.when(t + 2 < nbatch)
    def _():
      fire_stage(t + 2)

  fire_stage(0)
  drain_stage(0)
  if nbatch > 1:
    fire_stage(1)
  for d in gather_descs(0, bufa, sem_ga):
    d.start()

  def body(t, carry):
    @pl.when(t + 1 < nbatch)
    def _():
      drain_stage(t + 1)

    @pl.when(t % 2 == 0)
    def _():
      do_batch(t, bufa, sem_ga, sem_sa, bufb, sem_gb, sem_sb)

    @pl.when(t % 2 == 1)
    def _():
      do_batch(t, bufb, sem_gb, sem_sb, bufa, sem_ga, sem_sa)

    return carry

  lax.fori_loop(0, nbatch, body, 0)
  last = nbatch - 1
  lb, ls = (bufa, sem_sa) if last % 2 == 0 else (bufb, sem_sb)
  for d in scatter_descs(last, lb, ls):
    d.wait()


def _edge_scratch(d):
  return [
      pltpu.VMEM((3, BATCH), jnp.int32),
      pltpu.VMEM((3, K, CH), jnp.int32),
      pltpu.VMEM((BATCH, d), jnp.float32),
      pltpu.VMEM((BATCH, d), jnp.float32),
      pltpu.VMEM_SHARED((N, d), jnp.float32),
      pltpu.SemaphoreType.DMA,
      pltpu.SemaphoreType.DMA,
      pltpu.SemaphoreType.DMA,
      pltpu.SemaphoreType.DMA,
      pltpu.SemaphoreType.DMA,
  ]


@functools.partial(
    pl.kernel,
    mesh=_mesh,
    compiler_params=_sc_params,
    out_type=(
        jax.ShapeDtypeStruct((N, DH), jnp.float32),
        jax.ShapeDtypeStruct((N, DH), jnp.float32),
    ),
    scratch_types=_edge_scratch(DH),
)
def _agg64split(za_hbm, zb_hbm, src_hbm, dst2_hbm, zero_hbm, o0, o1,
                sidx3, didx3, bufa, bufb, acc_sh,
                sem_ga, sem_gb, sem_sa, sem_sb, sem_st):
  c = lax.axis_index("c")
  s = lax.axis_index("s")
  nbatch = E // NS // BATCH
  base = s * (NCHG // NS)
  _zero_acc(zero_hbm, acc_sh, s)
  plsc.subcore_barrier()

  @pl.when(c == 0)
  def _():
    _run_edge_pass(za_hbm, src_hbm, dst2_hbm, base, nbatch,
                   sidx3, didx3, bufa, bufb, acc_sh,
                   sem_ga, sem_gb, sem_sa, sem_sb, sem_st)

  @pl.when(c == 1)
  def _():
    _run_edge_pass(zb_hbm, src_hbm, dst2_hbm, base, nbatch,
                   sidx3, didx3, bufa, bufb, acc_sh,
                   sem_ga, sem_gb, sem_sa, sem_sb, sem_st)

  plsc.subcore_barrier()
  _copy_out(acc_sh, o0, o1, c, s)


@functools.partial(
    pl.kernel,
    mesh=_mesh,
    compiler_params=_sc_params,
    out_type=(
        jax.ShapeDtypeStruct((N, D2), jnp.float32),
        jax.ShapeDtypeStruct((N, D2), jnp.float32),
    ),
    scratch_types=_edge_scratch(D2),
)
def _agg16(z_hbm, src_hbm, dst2_hbm, zero_hbm, outa, outb,
           sidx3, didx3, bufa, bufb, acc_sh,
           sem_ga, sem_gb, sem_sa, sem_sb, sem_st):
  c = lax.axis_index("c")
  s = lax.axis_index("s")
  wid = s * NC + c
  nbatch = E // NW // BATCH
  base = wid * (NCHG // NW)
  _zero_acc(zero_hbm, acc_sh, s)
  plsc.subcore_barrier()
  _run_edge_pass(z_hbm, src_hbm, dst2_hbm, base, nbatch,
                 sidx3, didx3, bufa, bufb, acc_sh,
                 sem_ga, sem_gb, sem_sa, sem_sb, sem_st)
  plsc.subcore_barrier()
  _copy_out(acc_sh, outa, outb, c, s)


@functools.partial(
    pl.kernel,
    mesh=_mesh,
    compiler_params=_sc_params,
    out_type=(
        jax.ShapeDtypeStruct((N, D2), jnp.float32),
        jax.ShapeDtypeStruct((N, D2), jnp.float32),
    ),
    scratch_types=[
        pltpu.VMEM((E // NW // CH, CH), jnp.int32),
        pltpu.VMEM((CH, D2), jnp.float32),
        pltpu.VMEM_SHARED((N, D2), jnp.float32),
        pltpu.SemaphoreType.DMA,
        pltpu.SemaphoreType.DMA,
    ],
)
def _deg_kernel(dst2_hbm, ones_hbm, zero_hbm, outa, outb,
                didx_v, ones_v, acc_sh, sem_a, sem_b):
  c = lax.axis_index("c")
  s = lax.axis_index("s")
  wid = s * NC + c
  nchunk = E // NW // CH
  nb = nchunk // K
  pltpu.sync_copy(dst2_hbm.at[pl.ds(wid * nchunk, nchunk)], didx_v)
  pltpu.sync_copy(ones_hbm, ones_v)
  _zero_acc(zero_hbm, acc_sh, s)
  plsc.subcore_barrier()

  def descs(t, sem):
    return [pltpu.make_async_copy(ones_v, acc_sh.at[didx_v.at[t * K + b]],
                                  sem) for b in range(K)]

  def half(t, sem_p, sem_q):
    for b in range(K):
      pltpu.async_copy(ones_v, acc_sh.at[didx_v.at[t * K + b]], sem_p,
                       add=True)

    @pl.when(t > 0)
    def _():
      for d in descs(t - 1, sem_q):
        d.wait()

  def body(t, carry):
    @pl.when(t % 2 == 0)
    def _():
      half(t, sem_a, sem_b)

    @pl.when(t % 2 == 1)
    def _():
      half(t, sem_b, sem_a)

    return carry

  lax.fori_loop(0, nb, body, 0)
  for d in descs(nb - 1, sem_a if (nb - 1) % 2 == 0 else sem_b):
    d.wait()
  plsc.subcore_barrier()
  _copy_out(acc_sh, outa, outb, c, s)


BLK = 2000


def _z1_body(x_ref, w_ref, da_ref, db_ref, oa_ref, ob_ref):
  deg = da_ref[:, 0:1] + db_ref[:, 0:1] + 1.0
  dis = lax.rsqrt(deg)
  z = jnp.dot(x_ref[...], w_ref[...],
              preferred_element_type=jnp.float32) * dis
  oa_ref[...] = z[:, 0:DH]
  ob_ref[...] = z[:, DH:D1]


def _z2_body(s0_ref, s1_ref, za_ref, zb_ref,
             da_ref, db_ref, b1_ref, w2_ref, o_ref):
  deg = da_ref[:, 0:1] + db_ref[:, 0:1] + 1.0
  dis = lax.rsqrt(deg)
  hl = (s0_ref[...] + za_ref[...]) * dis + b1_ref[:, 0:DH]
  hh = (s1_ref[...] + zb_ref[...]) * dis + b1_ref[:, DH:D1]
  h = jnp.maximum(jnp.concatenate([hl, hh], axis=1), 0.0)
  o_ref[...] = jnp.dot(h, w2_ref[...], preferred_element_type=jnp.float32) * dis


def _out_body(sa_ref, sb_ref, z2_ref, da_ref, db_ref, b2_ref, o_ref):
  deg = da_ref[:, 0:1] + db_ref[:, 0:1] + 1.0
  dis = lax.rsqrt(deg)
  y = (sa_ref[...] + sb_ref[...] + z2_ref[...]) * dis
  o_ref[...] = y[:, 0:NCLS] + b2_ref[...]


def _row_spec(d):
  return pl.BlockSpec((BLK, d), lambda i: (i, 0))


def _full_spec(r, c):
  return pl.BlockSpec((r, c), lambda i: (0, 0))


def kernel(x, edge_index, W1, b1, W2, b2):
  src = edge_index[0].astype(jnp.int32)
  dst = edge_index[1].astype(jnp.int32)
  dst2 = dst.reshape(NCHG, CH)
  ones_rows = jnp.ones((CH, D2), jnp.float32)
  zeros16 = jnp.zeros((N, D2), jnp.float32)
  zeros64 = jnp.zeros((N, DH), jnp.float32)
  W2p = jnp.pad(W2, ((0, 0), (0, D2 - NCLS)))
  b1r = b1.reshape(1, D1)
  b2r = b2.reshape(1, NCLS)

  dega, degb = _deg_kernel(dst2, ones_rows, zeros16)

  z1a, z1b = pl.pallas_call(
      _z1_body,
      grid=(N // BLK,),
      in_specs=[_row_spec(D1), _full_spec(D1, D1), _row_spec(D2),
                _row_spec(D2)],
      out_specs=(_row_spec(DH), _row_spec(DH)),
      out_shape=(jax.ShapeDtypeStruct((N, DH), jnp.float32),
                 jax.ShapeDtypeStruct((N, DH), jnp.float32)),
  )(x, W1, dega, degb)

  s0, s1 = _agg64split(z1a, z1b, src, dst2, zeros64)

  z2 = pl.pallas_call(
      _z2_body,
      grid=(N // BLK,),
      in_specs=[_row_spec(DH)] * 4 + [_row_spec(D2), _row_spec(D2),
                _full_spec(1, D1), _full_spec(D1, D2)],
      out_specs=_row_spec(D2),
      out_shape=jax.ShapeDtypeStruct((N, D2), jnp.float32),
  )(s0, s1, z1a, z1b, dega, degb, b1r, W2p)

  s2a, s2b = _agg16(z2, src, dst2, zeros16)

  out = pl.pallas_call(
      _out_body,
      grid=(N // BLK,),
      in_specs=[_row_spec(D2), _row_spec(D2), _row_spec(D2), _row_spec(D2),
                _row_spec(D2), _full_spec(1, NCLS)],
      out_specs=_row_spec(NCLS),
      out_shape=jax.ShapeDtypeStruct((N, NCLS), jnp.float32),
  )(s2a, s2b, z2, dega, degb, b2r)

  return out
